# baseline (speedup 1.0000x reference)
import functools

import jax
import jax.numpy as jnp
from jax import lax
from jax.experimental import pallas as pl
from jax.experimental.pallas import tpu as pltpu
from jax.experimental.pallas import tpu_sc as plsc

_BATCH = 16384
_D = 64
_NW = 32
_RPT = _BATCH // _NW
_CHUNK = 128
_NCHUNK = _RPT // _CHUNK

_MARGIN = 1.0
_EPS2 = 0.1 * 0.1
_C = 0.25


def _row_hinge(bufs, r):
    def side(h_buf, t_buf, r_buf, n_buf):
        dot_v = jnp.zeros((16,), jnp.float32)
        ss_v = jnp.zeros((16,), jnp.float32)
        a = []
        n = []
        for j in range(_D // 16):
            sl = pl.ds(j * 16, 16)
            h = h_buf[r, sl]
            t = t_buf[r, sl]
            rr = r_buf[r, sl]
            nn = n_buf[r, sl]
            a.append(h + rr - t)
            n.append(nn)
            dot_v = dot_v + (h - t) * nn
            ss_v = ss_v + nn * nn
        dot = jnp.broadcast_to(jnp.sum(dot_v), (16,))
        ss = jnp.broadcast_to(jnp.sum(ss_v), (16,))
        f = dot / (ss + jnp.float32(1e-24))
        acc_v = jnp.abs(a[0] - f * n[0])
        for j in range(1, _D // 16):
            acc_v = acc_v + jnp.abs(a[j] - f * n[j])
        return jnp.sum(acc_v)

    ph, pt, pr, pn, nh, nt, nr, nn = bufs
    pos = side(ph, pt, pr, pn)
    neg = side(nh, nt, nr, nn)
    return jnp.maximum(pos - neg + jnp.float32(_MARGIN), jnp.float32(0.0))


def _sc_body(ih_hbm, it_hbm, ir_hbm, jh_hbm, jt_hbm, jr_hbm,
             ent_hbm, rel_hbm, norm_hbm, out_hbm,
             ih_v, it_v, ir_v, jh_v, jt_v, jr_v,
             b_ph, b_pt, b_pr, b_pn, b_nh, b_nt, b_nr, b_nn,
             out_v, sem):
    wid = lax.axis_index("s") * 2 + lax.axis_index("c")
    base = wid * _RPT

    pltpu.sync_copy(ih_hbm.at[pl.ds(base, _RPT)], ih_v)
    pltpu.sync_copy(it_hbm.at[pl.ds(base, _RPT)], it_v)
    pltpu.sync_copy(ir_hbm.at[pl.ds(base, _RPT)], ir_v)
    pltpu.sync_copy(jh_hbm.at[pl.ds(base, _RPT)], jh_v)
    pltpu.sync_copy(jt_hbm.at[pl.ds(base, _RPT)], jt_v)
    pltpu.sync_copy(jr_hbm.at[pl.ds(base, _RPT)], jr_v)

    bufs = (b_ph, b_pt, b_pr, b_pn, b_nh, b_nt, b_nr, b_nn)
    acc = jnp.float32(0.0)
    for k in range(_NCHUNK):
        sl = pl.ds(k * _CHUNK, _CHUNK)
        copies = [
            pltpu.async_copy(ent_hbm.at[ih_v.at[sl]], b_ph, sem),
            pltpu.async_copy(ent_hbm.at[it_v.at[sl]], b_pt, sem),
            pltpu.async_copy(rel_hbm.at[ir_v.at[sl]], b_pr, sem),
            pltpu.async_copy(norm_hbm.at[ir_v.at[sl]], b_pn, sem),
            pltpu.async_copy(ent_hbm.at[jh_v.at[sl]], b_nh, sem),
            pltpu.async_copy(ent_hbm.at[jt_v.at[sl]], b_nt, sem),
            pltpu.async_copy(rel_hbm.at[jr_v.at[sl]], b_nr, sem),
            pltpu.async_copy(norm_hbm.at[jr_v.at[sl]], b_nn, sem),
        ]
        for c in copies:
            c.wait()

        def body(r, a):
            return a + _row_hinge(bufs, r)

        acc = lax.fori_loop(0, _CHUNK, body, acc)

    lane = lax.iota(jnp.int32, 16)
    out_v[...] = jnp.where(lane == 0, acc, jnp.float32(0.0))
    pltpu.sync_copy(out_v, out_hbm.at[wid])


@jax.jit
def _sc_hinge(ih, it, ir, jh, jt, jr, ent_table, rel_table, norm_table):
    mesh = plsc.VectorSubcoreMesh(core_axis_name="c", subcore_axis_name="s")
    f = pl.kernel(
        _sc_body,
        mesh=mesh,
        compiler_params=pltpu.CompilerParams(
            needs_layout_passes=False, use_tc_tiling_on_sc=False),
        out_type=jax.ShapeDtypeStruct((_NW, 16), jnp.float32),
        scratch_types=[
            pltpu.VMEM((_RPT,), jnp.int32),
            pltpu.VMEM((_RPT,), jnp.int32),
            pltpu.VMEM((_RPT,), jnp.int32),
            pltpu.VMEM((_RPT,), jnp.int32),
            pltpu.VMEM((_RPT,), jnp.int32),
            pltpu.VMEM((_RPT,), jnp.int32),
            pltpu.VMEM((_CHUNK, _D), jnp.float32),
            pltpu.VMEM((_CHUNK, _D), jnp.float32),
            pltpu.VMEM((_CHUNK, _D), jnp.float32),
            pltpu.VMEM((_CHUNK, _D), jnp.float32),
            pltpu.VMEM((_CHUNK, _D), jnp.float32),
            pltpu.VMEM((_CHUNK, _D), jnp.float32),
            pltpu.VMEM((_CHUNK, _D), jnp.float32),
            pltpu.VMEM((_CHUNK, _D), jnp.float32),
            pltpu.VMEM((16,), jnp.float32),
            pltpu.SemaphoreType.DMA,
        ],
    )
    return f(ih, it, ir, jh, jt, jr, ent_table, rel_table, norm_table)


_EBLK = 32768
_N_ENT = 1000000


def _reg_body(entt_ref, relt_ref, normt_ref, out_ref, acc_ref):
    i = pl.program_id(0)

    @pl.when(i == 0)
    def _init():
        acc_ref[0] = jnp.float32(0.0)

    blk = entt_ref[...]
    ss = jnp.sum(blk * blk, axis=0, keepdims=True)
    col = i * _EBLK + jax.lax.broadcasted_iota(jnp.int32, (1, _EBLK), 1)
    norms = jnp.where(col < _N_ENT, jnp.sqrt(ss), jnp.float32(0.0))
    acc_ref[0] += jnp.sum(norms)

    @pl.when(i == pl.num_programs(0) - 1)
    def _final():
        rel = relt_ref[...]
        nrm = normt_ref[...]
        rn = jnp.sum(jnp.sum(nrm, axis=0) * jnp.sum(rel, axis=0))
        den = jnp.sum(jnp.sqrt(jnp.sum(rel * rel, axis=0)))
        rel_loss = jnp.maximum(rn / den - jnp.float32(_EPS2), jnp.float32(0.0))
        ent_loss = jnp.maximum(acc_ref[0] - jnp.float32(1.0), jnp.float32(0.0))
        out_ref[0, 0] = jnp.float32(_C) * (ent_loss + rel_loss)


@jax.jit
def _tc_reg(entt, relt, normt):
    n_ent = entt.shape[1]
    grid = ((n_ent + _EBLK - 1) // _EBLK,)
    return pl.pallas_call(
        _reg_body,
        grid=grid,
        in_specs=[
            pl.BlockSpec((_D, _EBLK), lambda i: (0, i)),
            pl.BlockSpec((_D, relt.shape[1]), lambda i: (0, 0)),
            pl.BlockSpec((_D, normt.shape[1]), lambda i: (0, 0)),
        ],
        out_specs=pl.BlockSpec(memory_space=pltpu.SMEM),
        out_shape=jax.ShapeDtypeStruct((1, 1), jnp.float32),
        scratch_shapes=[pltpu.SMEM((1,), jnp.float32)],
    )(entt, relt, normt)


def kernel(x, ent_table, rel_table, norm_table):
    ih = x[:, 0]
    it = x[:, 1]
    ir = x[:, 2]
    jh = x[:, 3]
    jt = x[:, 4]
    jr = x[:, 5]
    hinge_parts = _sc_hinge(ih, it, ir, jh, jt, jr,
                            ent_table, rel_table, norm_table)
    reg = _tc_reg(ent_table.T, rel_table.T, norm_table.T)
    return jnp.sum(hinge_parts) + reg[0, 0]


# speedup vs baseline: 2.3448x; 2.3448x over previous
import functools

import jax
import jax.numpy as jnp
from jax import lax
from jax.experimental import pallas as pl
from jax.experimental.pallas import tpu as pltpu
from jax.experimental.pallas import tpu_sc as plsc

_BATCH = 16384
_D = 64
_NW = 32
_RPT = _BATCH // _NW
_CHUNK = 128
_NCHUNK = _RPT // _CHUNK

_MARGIN = 1.0
_EPS2 = 0.1 * 0.1
_C = 0.25

_N_ENT = 1000000
_EBLK = 32768


def _scan_repack_body(entt_ref, relt_ref, normt_ref, entp_ref, out_ref,
                      acc_ref):
    i = pl.program_id(0)

    @pl.when(i == 0)
    def _init():
        acc_ref[0] = jnp.float32(0.0)

    blk = entt_ref[...]
    ss = jnp.sum(blk * blk, axis=0, keepdims=True)
    col = i * _EBLK + jax.lax.broadcasted_iota(jnp.int32, (1, _EBLK), 1)
    norms = jnp.where(col < _N_ENT, jnp.sqrt(ss), jnp.float32(0.0))
    acc_ref[0] += jnp.sum(norms)

    t = blk.T
    entp_ref[:, 0:_D] = t
    entp_ref[:, _D:2 * _D] = jnp.zeros((_EBLK, _D), jnp.float32)

    @pl.when(i == pl.num_programs(0) - 1)
    def _final():
        rel = relt_ref[...]
        nrm = normt_ref[...]
        rn = jnp.sum(jnp.sum(nrm, axis=0) * jnp.sum(rel, axis=0))
        den = jnp.sum(jnp.sqrt(jnp.sum(rel * rel, axis=0)))
        rel_loss = jnp.maximum(rn / den - jnp.float32(_EPS2), jnp.float32(0.0))
        ent_loss = jnp.maximum(acc_ref[0] - jnp.float32(1.0), jnp.float32(0.0))
        out_ref[0, 0] = jnp.float32(_C) * (ent_loss + rel_loss)


@jax.jit
def _tc_scan_repack(entt, relt, normt):
    n_ent = entt.shape[1]
    grid = ((n_ent + _EBLK - 1) // _EBLK,)
    return pl.pallas_call(
        _scan_repack_body,
        grid=grid,
        in_specs=[
            pl.BlockSpec((_D, _EBLK), lambda i: (0, i)),
            pl.BlockSpec((_D, relt.shape[1]), lambda i: (0, 0)),
            pl.BlockSpec((_D, normt.shape[1]), lambda i: (0, 0)),
        ],
        out_specs=[
            pl.BlockSpec((_EBLK, 2 * _D), lambda i: (i, 0)),
            pl.BlockSpec(memory_space=pltpu.SMEM),
        ],
        out_shape=[
            jax.ShapeDtypeStruct((n_ent, 2 * _D), jnp.float32),
            jax.ShapeDtypeStruct((1, 1), jnp.float32),
        ],
        scratch_shapes=[pltpu.SMEM((1,), jnp.float32)],
    )(entt, relt, normt)


def _row_hinge(bufs, r):
    def side(h_buf, t_buf, rn_buf):
        dot_v = jnp.zeros((16,), jnp.float32)
        ss_v = jnp.zeros((16,), jnp.float32)
        a = []
        n = []
        for j in range(_D // 16):
            sl = pl.ds(j * 16, 16)
            sln = pl.ds(_D + j * 16, 16)
            h = h_buf[r, sl]
            t = t_buf[r, sl]
            rr = rn_buf[r, sl]
            nn = rn_buf[r, sln]
            a.append(h + rr - t)
            n.append(nn)
            dot_v = dot_v + (h - t) * nn
            ss_v = ss_v + nn * nn
        dot = jnp.broadcast_to(jnp.sum(dot_v), (16,))
        ss = jnp.broadcast_to(jnp.sum(ss_v), (16,))
        f = dot / (ss + jnp.float32(1e-24))
        acc_v = jnp.abs(a[0] - f * n[0])
        for j in range(1, _D // 16):
            acc_v = acc_v + jnp.abs(a[j] - f * n[j])
        return jnp.sum(acc_v)

    ph, pt, prn, nh, nt, nrn = bufs
    pos = side(ph, pt, prn)
    neg = side(nh, nt, nrn)
    return jnp.maximum(pos - neg + jnp.float32(_MARGIN), jnp.float32(0.0))


def _sc_body(ih_hbm, it_hbm, ir_hbm, jh_hbm, jt_hbm, jr_hbm,
             entp_hbm, relnorm_hbm, out_hbm,
             ih_v, it_v, ir_v, jh_v, jt_v, jr_v,
             b_ph, b_pt, b_prn, b_nh, b_nt, b_nrn,
             out_v, sem):
    wid = lax.axis_index("s") * 2 + lax.axis_index("c")
    base = wid * _RPT

    pltpu.sync_copy(ih_hbm.at[pl.ds(base, _RPT)], ih_v)
    pltpu.sync_copy(it_hbm.at[pl.ds(base, _RPT)], it_v)
    pltpu.sync_copy(ir_hbm.at[pl.ds(base, _RPT)], ir_v)
    pltpu.sync_copy(jh_hbm.at[pl.ds(base, _RPT)], jh_v)
    pltpu.sync_copy(jt_hbm.at[pl.ds(base, _RPT)], jt_v)
    pltpu.sync_copy(jr_hbm.at[pl.ds(base, _RPT)], jr_v)

    bufs = (b_ph, b_pt, b_prn, b_nh, b_nt, b_nrn)
    acc = jnp.float32(0.0)
    for k in range(_NCHUNK):
        sl = pl.ds(k * _CHUNK, _CHUNK)
        copies = [
            pltpu.async_copy(entp_hbm.at[ih_v.at[sl]], b_ph, sem),
            pltpu.async_copy(entp_hbm.at[it_v.at[sl]], b_pt, sem),
            pltpu.async_copy(relnorm_hbm.at[ir_v.at[sl]], b_prn, sem),
            pltpu.async_copy(entp_hbm.at[jh_v.at[sl]], b_nh, sem),
            pltpu.async_copy(entp_hbm.at[jt_v.at[sl]], b_nt, sem),
            pltpu.async_copy(relnorm_hbm.at[jr_v.at[sl]], b_nrn, sem),
        ]
        for c in copies:
            c.wait()

        def body(r, a):
            return a + _row_hinge(bufs, r)

        acc = lax.fori_loop(0, _CHUNK, body, acc)

    lane = lax.iota(jnp.int32, 16)
    out_v[...] = jnp.where(lane == 0, acc, jnp.float32(0.0))
    pltpu.sync_copy(out_v, out_hbm.at[pl.ds(wid * 16, 16)])


@jax.jit
def _sc_hinge(ih, it, ir, jh, jt, jr, entp, relnorm):
    mesh = plsc.VectorSubcoreMesh(core_axis_name="c", subcore_axis_name="s")
    f = pl.kernel(
        _sc_body,
        mesh=mesh,
        compiler_params=pltpu.CompilerParams(needs_layout_passes=False),
        out_type=jax.ShapeDtypeStruct((_NW * 16,), jnp.float32),
        scratch_types=[
            pltpu.VMEM((_RPT,), jnp.int32),
            pltpu.VMEM((_RPT,), jnp.int32),
            pltpu.VMEM((_RPT,), jnp.int32),
            pltpu.VMEM((_RPT,), jnp.int32),
            pltpu.VMEM((_RPT,), jnp.int32),
            pltpu.VMEM((_RPT,), jnp.int32),
            pltpu.VMEM((_CHUNK, 2 * _D), jnp.float32),
            pltpu.VMEM((_CHUNK, 2 * _D), jnp.float32),
            pltpu.VMEM((_CHUNK, 2 * _D), jnp.float32),
            pltpu.VMEM((_CHUNK, 2 * _D), jnp.float32),
            pltpu.VMEM((_CHUNK, 2 * _D), jnp.float32),
            pltpu.VMEM((_CHUNK, 2 * _D), jnp.float32),
            pltpu.VMEM((16,), jnp.float32),
            pltpu.SemaphoreType.DMA,
        ],
    )
    return f(ih, it, ir, jh, jt, jr, entp, relnorm)


def kernel(x, ent_table, rel_table, norm_table):
    entp, reg = _tc_scan_repack(ent_table.T, rel_table.T, norm_table.T)
    relnorm = jnp.concatenate([rel_table, norm_table], axis=1)
    hinge_parts = _sc_hinge(x[:, 0], x[:, 1], x[:, 2], x[:, 3], x[:, 4],
                            x[:, 5], entp, relnorm)
    return jnp.sum(hinge_parts) + reg[0, 0]
